# baseline (speedup 1.0000x reference)
import functools

import jax
import jax.numpy as jnp
from jax import lax
from jax.experimental import pallas as pl
from jax.experimental.pallas import tpu as pltpu
from jax.experimental.pallas import tpu_sc as plsc

_NUM_SPECIAL = 16
_HALF = 128


def _mask_body(idx_ref, m_ref):
    S = m_ref.shape[1]
    pos = lax.broadcasted_iota(jnp.int32, (1, S), 1)
    m = jnp.zeros((1, S), dtype=jnp.bool_)
    for j in range(_NUM_SPECIAL):
        m = jnp.logical_or(m, pos == idx_ref[j])
    m_ref[...] = m.astype(jnp.int32)


def _sc_body(w_hbm, mask_hbm, out_hbm, mask_v, rows_v, sem, B, S, H):
    info = plsc.get_sparse_core_info()
    nc = info.num_cores
    wid = lax.axis_index("s") * nc + lax.axis_index("c")
    nw = nc * info.num_subcores
    chunk = S // nw

    for h in range(chunk // _HALF):
        sub = wid * chunk + h * _HALF
        pltpu.sync_copy(mask_hbm.at[pl.ds(sub, _HALF)], mask_v)
        pltpu.async_copy(w_hbm.at[mask_v], rows_v, sem).wait()
        for b in range(B):
            pltpu.sync_copy(rows_v, out_hbm.at[pl.ds(b * S + sub, _HALF)])


def kernel(x, special_tokens_indices, W):
    B, S, H = x.shape
    idx = special_tokens_indices.astype(jnp.int32)

    mask = pl.pallas_call(
        _mask_body,
        grid=(1,),
        in_specs=[pl.BlockSpec(memory_space=pltpu.SMEM)],
        out_specs=pl.BlockSpec((1, S), lambda i: (0, 0)),
        out_shape=jax.ShapeDtypeStruct((1, S), jnp.int32),
    )(idx)
    mask = mask.reshape(S)

    k = functools.partial(
        pl.kernel,
        mesh=plsc.VectorSubcoreMesh(core_axis_name="c", subcore_axis_name="s"),
        out_type=jax.ShapeDtypeStruct((B * S, H), jnp.float32),
        scratch_types=[
            pltpu.VMEM((_HALF,), jnp.int32),
            pltpu.VMEM((_HALF, H), jnp.float32),
            pltpu.SemaphoreType.DMA,
        ],
    )(functools.partial(_sc_body, B=B, S=S, H=H))
    out = k(W, mask)
    return out.reshape(B, S, H)
